# baseline (speedup 1.0000x reference)
import dataclasses
import functools

import jax
import jax.numpy as jnp
from jax import lax
from jax.experimental import pallas as pl
from jax.experimental.pallas import tpu as pltpu
from jax.experimental.pallas import tpu_sc as plsc

_SIZE = 1048576
_P = 2147483647
_CHUNK = 8
_NHASH = 8
_BATCH = 4096
_FEAT = 26
_M = _BATCH * _FEAT
_NCHUNKS = _M * _NHASH
_NW = 32
_CPW = _NCHUNKS // _NW
_XPW = _M // _NW
_NB = 4
_K = _CPW // _NB
_KR = _K // 128

_CP = pltpu.CompilerParams()
if "needs_layout_passes" in pltpu.CompilerParams.__dataclass_fields__:
    _CP = dataclasses.replace(_CP, needs_layout_passes=False)
if "use_tc_tiling_on_sc" in pltpu.CompilerParams.__dataclass_fields__:
    _CP = dataclasses.replace(_CP, use_tc_tiling_on_sc=False)


@functools.partial(
    pl.kernel,
    out_type=jax.ShapeDtypeStruct((_NCHUNKS // 128, 128, _CHUNK), jnp.float32),
    mesh=plsc.VectorSubcoreMesh(core_axis_name="c", subcore_axis_name="s"),
    scratch_types=[
        pltpu.VMEM((_XPW,), jnp.int32),
        pltpu.VMEM((16,), jnp.int32),
        pltpu.VMEM((_KR, 128), jnp.int32),
        pltpu.VMEM((_KR, 128, _CHUNK), jnp.float32),
        pltpu.SemaphoreType.DMA,
    ],
    compiler_params=_CP,
)
def _robe_sc(x_hbm, w_hbm, ab_hbm, out_hbm, x_v, ab_v, idx_v, rows_v, sem):
    w2d = w_hbm
    out3d = out_hbm
    wid = lax.axis_index("s") * 2 + lax.axis_index("c")
    pltpu.sync_copy(x_hbm.at[pl.ds(wid * _XPW, _XPW)], x_v)
    pltpu.sync_copy(ab_hbm, ab_v)
    lane = lax.iota(jnp.int32, 16)
    k8 = lane & 7
    av = plsc.load_gather(ab_v, [k8])
    bv = plsc.load_gather(ab_v, [k8 + 8])

    @pl.loop(0, _NB)
    def _blk(blk):
        cbase = blk * _K

        @pl.loop(0, _KR)
        def _hash_row(j):
            @pl.loop(0, 8)
            def _grp(q):
                c = (cbase + j * 128 + q * 16) + lane
                xv = plsc.load_gather(x_v, [c >> 3])
                v = xv * av + bv
                v = jnp.where(v < 0, v + _P, v)
                v = jnp.where(v < 0, v + _P, v)
                v = jnp.where(v >= _P, v - _P, v)
                h = v & (_SIZE - 1)
                idx_v[j, pl.ds(q * 16, 16)] = ((h & 7) << 17) + (h >> 3)

        @pl.loop(0, _KR)
        def _fire(j):
            pltpu.async_copy(w2d.at[idx_v.at[j]], rows_v.at[j], sem)

        @pl.loop(0, _KR)
        def _drain(j):
            pltpu.make_async_copy(w2d.at[idx_v.at[j]], rows_v.at[j], sem).wait()

        pltpu.sync_copy(
            rows_v, out3d.at[pl.ds(wid * (_CPW // 128) + blk * _KR, _KR)]
        )


def kernel(input_tensor, data, a, b):
    x_flat = input_tensor.reshape(-1)
    data_ext = jnp.concatenate([data, jnp.broadcast_to(data[-1], (_CHUNK,))])
    w = jnp.concatenate(
        [data_ext[s:s + _SIZE] for s in range(_CHUNK)]
    ).reshape(_SIZE, _CHUNK)
    ab = jnp.concatenate([a, b])
    out = _robe_sc(x_flat, w, ab)
    return out.reshape(_BATCH, _FEAT, _NHASH * _CHUNK)


# speedup vs baseline: 1.7886x; 1.7886x over previous
import dataclasses
import functools

import jax
import jax.numpy as jnp
from jax import lax
from jax.experimental import pallas as pl
from jax.experimental.pallas import tpu as pltpu
from jax.experimental.pallas import tpu_sc as plsc

_SIZE = 1048576
_P = 2147483647
_CHUNK = 8
_NHASH = 8
_BATCH = 4096
_FEAT = 26
_M = _BATCH * _FEAT
_NCHUNKS = _M * _NHASH
_NW = 32
_CPW = _NCHUNKS // _NW
_XPW = _M // _NW
_NB = 4
_K = _CPW // _NB
_KR = _K // 128

_CP = pltpu.CompilerParams()
if "needs_layout_passes" in pltpu.CompilerParams.__dataclass_fields__:
    _CP = dataclasses.replace(_CP, needs_layout_passes=False)
if "use_tc_tiling_on_sc" in pltpu.CompilerParams.__dataclass_fields__:
    _CP = dataclasses.replace(_CP, use_tc_tiling_on_sc=False)


@functools.partial(
    pl.kernel,
    out_type=jax.ShapeDtypeStruct((_NCHUNKS // 128, 128, _CHUNK), jnp.float32),
    mesh=plsc.VectorSubcoreMesh(core_axis_name="c", subcore_axis_name="s"),
    scratch_types=[
        pltpu.VMEM((_XPW,), jnp.int32),
        pltpu.VMEM((16,), jnp.int32),
        pltpu.VMEM((_KR, 128), jnp.int32),
        pltpu.VMEM((_KR, 128, _CHUNK), jnp.float32),
        pltpu.SemaphoreType.DMA,
    ],
    compiler_params=_CP,
)
def _robe_sc(x_hbm, w_hbm, ab_hbm, out_hbm, x_v, ab_v, idx_v, rows_v, sem):
    w2d = w_hbm
    out3d = out_hbm
    wid = lax.axis_index("s") * 2 + lax.axis_index("c")
    pltpu.sync_copy(x_hbm.at[pl.ds(wid * _XPW, _XPW)], x_v)
    pltpu.sync_copy(ab_hbm, ab_v)
    lane = lax.iota(jnp.int32, 16)
    k8 = lane & 7
    av = plsc.load_gather(ab_v, [k8])
    bv = plsc.load_gather(ab_v, [k8 + 8])

    @pl.loop(0, _NB)
    def _blk(blk):
        cbase = blk * _K

        @pl.loop(0, _KR)
        def _hash_row(j):
            @pl.loop(0, 8)
            def _grp(q):
                c = (cbase + j * 128 + q * 16) + lane
                xv = plsc.load_gather(x_v, [c >> 3])
                v = xv * av + bv
                v = jnp.where(v < 0, v + _P, v)
                v = jnp.where(v < 0, v + _P, v)
                v = jnp.where(v >= _P, v - _P, v)
                h = v & (_SIZE - 1)
                idx_v[j, pl.ds(q * 16, 16)] = h

        @pl.loop(0, _KR)
        def _fire(j):
            pltpu.async_copy(w2d.at[idx_v.at[j]], rows_v.at[j], sem)

        @pl.loop(0, _KR)
        def _drain(j):
            pltpu.make_async_copy(w2d.at[idx_v.at[j]], rows_v.at[j], sem).wait()

        pltpu.sync_copy(
            rows_v, out3d.at[pl.ds(wid * (_CPW // 128) + blk * _KR, _KR)]
        )


def kernel(input_tensor, data, a, b):
    x_flat = input_tensor.reshape(-1)
    data_ext = jnp.concatenate([data, jnp.broadcast_to(data[-1], (_CHUNK,))])
    w = jnp.stack([data_ext[s:s + _SIZE] for s in range(_CHUNK)], axis=0).T
    ab = jnp.concatenate([a, b])
    out = _robe_sc(x_flat, w, ab)
    return out.reshape(_BATCH, _FEAT, _NHASH * _CHUNK)


# speedup vs baseline: 4.7167x; 2.6371x over previous
import dataclasses
import functools

import jax
import jax.numpy as jnp
from jax import lax
from jax.experimental import pallas as pl
from jax.experimental.pallas import tpu as pltpu
from jax.experimental.pallas import tpu_sc as plsc

_SIZE = 1048576
_P = 2147483647
_CHUNK = 8
_NHASH = 8
_BATCH = 4096
_FEAT = 26
_M = _BATCH * _FEAT
_NCHUNKS = _M * _NHASH
_NW = 32
_CPW = _NCHUNKS // _NW
_XPW = _M // _NW
_NB = 4
_K = _CPW // _NB
_KR = _K // 128
_WPW = _SIZE * _CHUNK // _NW
_NSB = 8
_SB = _WPW // _NSB
_SBR = _SB // _CHUNK

_CP = pltpu.CompilerParams()
if "needs_layout_passes" in pltpu.CompilerParams.__dataclass_fields__:
    _CP = dataclasses.replace(_CP, needs_layout_passes=False)
if "use_tc_tiling_on_sc" in pltpu.CompilerParams.__dataclass_fields__:
    _CP = dataclasses.replace(_CP, use_tc_tiling_on_sc=False)

_MESH = plsc.VectorSubcoreMesh(core_axis_name="c", subcore_axis_name="s")


@functools.partial(
    pl.kernel,
    out_type=jax.ShapeDtypeStruct((_SIZE, _CHUNK), jnp.float32),
    mesh=_MESH,
    scratch_types=[
        pltpu.VMEM((_SB + 8,), jnp.float32),
        pltpu.VMEM((_SBR, _CHUNK), jnp.float32),
        pltpu.VMEM((_SBR, _CHUNK), jnp.float32),
        pltpu.SemaphoreType.DMA,
    ],
    compiler_params=_CP,
)
def _build_w_sc(dext_hbm, w_hbm, buf_v, sh_a, sh_b, sem):
    wid = lax.axis_index("s") * 2 + lax.axis_index("c")
    s = wid >> 2
    q = wid & 3
    src0 = q * _WPW
    row0 = wid * (_WPW // _CHUNK)
    lane = lax.iota(jnp.int32, 16)
    rowpat = lane >> 3
    colpat = lane & 7
    shs = (sh_a, sh_b)

    def fill(sb, sh_v):
        pltpu.sync_copy(dext_hbm.at[pl.ds(src0 + sb * _SB, _SB + 8)], buf_v)

        @pl.loop(0, _SB // 16)
        def _shift(i):
            v = plsc.load_gather(buf_v, [(i * 16 + s) + lane])
            plsc.store_scatter(sh_v, [(2 * i) + rowpat, colpat], v)

    def wout_start(sb, sh_v):
        pltpu.async_copy(sh_v, w_hbm.at[pl.ds(row0 + sb * _SBR, _SBR)], sem)

    def wout_wait(sb, sh_v):
        pltpu.make_async_copy(
            sh_v, w_hbm.at[pl.ds(row0 + sb * _SBR, _SBR)], sem
        ).wait()

    for sb in range(_NSB):
        b = sb & 1
        if sb >= 2:
            wout_wait(sb - 2, shs[b])
        fill(sb, shs[b])
        wout_start(sb, shs[b])
    wout_wait(_NSB - 2, shs[0])
    wout_wait(_NSB - 1, shs[1])


@functools.partial(
    pl.kernel,
    out_type=jax.ShapeDtypeStruct((_NCHUNKS // 128, 128, _CHUNK), jnp.float32),
    mesh=_MESH,
    scratch_types=[
        pltpu.VMEM((_XPW,), jnp.int32),
        pltpu.VMEM((16,), jnp.int32),
        pltpu.VMEM((_KR, 128), jnp.int32),
        pltpu.VMEM((_KR, 128, _CHUNK), jnp.float32),
        pltpu.SemaphoreType.DMA,
    ],
    compiler_params=_CP,
)
def _robe_sc(x_hbm, w_hbm, ab_hbm, out_hbm, x_v, ab_v, idx_v, rows_v, sem):
    wid = lax.axis_index("s") * 2 + lax.axis_index("c")
    pltpu.sync_copy(x_hbm.at[pl.ds(wid * _XPW, _XPW)], x_v)
    pltpu.sync_copy(ab_hbm, ab_v)
    lane = lax.iota(jnp.int32, 16)
    k8 = lane & 7
    av = plsc.load_gather(ab_v, [k8])
    bv = plsc.load_gather(ab_v, [k8 + 8])

    @pl.loop(0, _NB)
    def _blk(blk):
        cbase = blk * _K

        @pl.loop(0, _KR)
        def _hash_row(j):
            @pl.loop(0, 8)
            def _grp(q):
                c = (cbase + j * 128 + q * 16) + lane
                xv = plsc.load_gather(x_v, [c >> 3])
                v = xv * av + bv
                v = jnp.where(v < 0, v + _P, v)
                v = jnp.where(v < 0, v + _P, v)
                v = jnp.where(v >= _P, v - _P, v)
                h = v & (_SIZE - 1)
                idx_v[j, pl.ds(q * 16, 16)] = ((h & 7) << 17) + (h >> 3)

        @pl.loop(0, _KR)
        def _fire(j):
            pltpu.async_copy(w_hbm.at[idx_v.at[j]], rows_v.at[j], sem)

        @pl.loop(0, _KR)
        def _drain(j):
            pltpu.make_async_copy(w_hbm.at[idx_v.at[j]], rows_v.at[j], sem).wait()

        pltpu.sync_copy(
            rows_v, out_hbm.at[pl.ds(wid * (_CPW // 128) + blk * _KR, _KR)]
        )


def kernel(input_tensor, data, a, b):
    x_flat = input_tensor.reshape(-1)
    data_ext = jnp.concatenate([data, jnp.broadcast_to(data[-1], (_CHUNK,))])
    w = _build_w_sc(data_ext)
    ab = jnp.concatenate([a, b])
    out = _robe_sc(x_flat, w, ab)
    return out.reshape(_BATCH, _FEAT, _NHASH * _CHUNK)


# speedup vs baseline: 4.9886x; 1.0576x over previous
import dataclasses
import functools

import jax
import jax.numpy as jnp
from jax import lax
from jax.experimental import pallas as pl
from jax.experimental.pallas import tpu as pltpu
from jax.experimental.pallas import tpu_sc as plsc

_SIZE = 1048576
_P = 2147483647
_CHUNK = 8
_NHASH = 8
_BATCH = 4096
_FEAT = 26
_M = _BATCH * _FEAT
_NCHUNKS = _M * _NHASH
_NW = 32
_CPW = _NCHUNKS // _NW
_RPW = _BATCH // _NW
_NB = 4
_RPB = _RPW // _NB
_K = _CPW // _NB
_KR = _K // 128
_GPR = _FEAT * _NHASH // 16
_WPW = _SIZE * _CHUNK // _NW
_NSB = 8
_SB = _WPW // _NSB
_SBR = _SB // _CHUNK

_CP = pltpu.CompilerParams()
if "needs_layout_passes" in pltpu.CompilerParams.__dataclass_fields__:
    _CP = dataclasses.replace(_CP, needs_layout_passes=False)
if "use_tc_tiling_on_sc" in pltpu.CompilerParams.__dataclass_fields__:
    _CP = dataclasses.replace(_CP, use_tc_tiling_on_sc=False)

_MESH = plsc.VectorSubcoreMesh(core_axis_name="c", subcore_axis_name="s")


@functools.partial(
    pl.kernel,
    out_type=jax.ShapeDtypeStruct((_SIZE, _CHUNK), jnp.float32),
    mesh=_MESH,
    scratch_types=[
        pltpu.VMEM((_SB + 8,), jnp.float32),
        pltpu.VMEM((_SBR, _CHUNK), jnp.float32),
        pltpu.VMEM((_SBR, _CHUNK), jnp.float32),
        pltpu.SemaphoreType.DMA,
    ],
    compiler_params=_CP,
)
def _build_w_sc(dext_hbm, w_hbm, buf_v, sh_a, sh_b, sem):
    wid = lax.axis_index("s") * 2 + lax.axis_index("c")
    s = wid >> 2
    q = wid & 3
    src0 = q * _WPW
    row0 = wid * (_WPW // _CHUNK)
    lane = lax.iota(jnp.int32, 16)
    rowpat = lane >> 3
    colpat = lane & 7
    shs = (sh_a, sh_b)

    def fill(sb, sh_v):
        pltpu.sync_copy(dext_hbm.at[pl.ds(src0 + sb * _SB, _SB + 8)], buf_v)

        @pl.loop(0, _SB // 16)
        def _shift(i):
            v = plsc.load_gather(buf_v, [(i * 16 + s) + lane])
            plsc.store_scatter(sh_v, [(2 * i) + rowpat, colpat], v)

    def wout_start(sb, sh_v):
        pltpu.async_copy(sh_v, w_hbm.at[pl.ds(row0 + sb * _SBR, _SBR)], sem)

    def wout_wait(sb, sh_v):
        pltpu.make_async_copy(
            sh_v, w_hbm.at[pl.ds(row0 + sb * _SBR, _SBR)], sem
        ).wait()

    for sb in range(_NSB):
        b = sb & 1
        if sb >= 2:
            wout_wait(sb - 2, shs[b])
        fill(sb, shs[b])
        wout_start(sb, shs[b])
    wout_wait(_NSB - 2, shs[0])
    wout_wait(_NSB - 1, shs[1])


@functools.partial(
    pl.kernel,
    out_type=jax.ShapeDtypeStruct((_NCHUNKS // 128, 128, _CHUNK), jnp.float32),
    mesh=_MESH,
    scratch_types=[
        pltpu.VMEM((_RPW, _FEAT), jnp.int32),
        pltpu.VMEM((16,), jnp.int32),
        pltpu.VMEM((_KR, 128), jnp.int32),
        pltpu.VMEM((_KR, 128), jnp.int32),
        pltpu.VMEM((_KR, 128, _CHUNK), jnp.float32),
        pltpu.VMEM((_KR, 128, _CHUNK), jnp.float32),
        pltpu.SemaphoreType.DMA,
        pltpu.SemaphoreType.DMA,
    ],
    compiler_params=_CP,
)
def _robe_sc(x_hbm, w_hbm, ab_hbm, out_hbm,
             x_v, ab_v, idx0, idx1, rows0, rows1, gsem, osem):
    wid = lax.axis_index("s") * 2 + lax.axis_index("c")
    pltpu.sync_copy(x_hbm.at[pl.ds(wid * _RPW, _RPW)], x_v)
    pltpu.sync_copy(ab_hbm, ab_v)
    lane = lax.iota(jnp.int32, 16)
    k8 = lane & 7
    av = plsc.load_gather(ab_v, [k8])
    bv = plsc.load_gather(ab_v, [k8 + 8])
    idxs = (idx0, idx1)
    rows = (rows0, rows1)

    def hash_block(blk, idx_v):
        @pl.loop(0, _RPB)
        def _row(r2):
            r = blk * _RPB + r2

            @pl.loop(0, _GPR)
            def _grp(g):
                col = (g * 16 + lane) >> 3
                xv = plsc.load_gather(x_v, [jnp.full((16,), r, jnp.int32), col])
                v = xv * av + bv
                v = jnp.where(v < 0, v + _P, v)
                v = jnp.where(v < 0, v + _P, v)
                v = jnp.where(v >= _P, v - _P, v)
                h = v & (_SIZE - 1)
                grp = r2 * _GPR + g
                idx_v[grp >> 3, pl.ds((grp & 7) * 16, 16)] = (
                    ((h & 7) << 17) + (h >> 3)
                )

    def fire(idx_v, rows_v):
        @pl.loop(0, _KR)
        def _f(j):
            pltpu.async_copy(w_hbm.at[idx_v.at[j]], rows_v.at[j], gsem)

    def drain(idx_v, rows_v):
        @pl.loop(0, _KR)
        def _d(j):
            pltpu.make_async_copy(w_hbm.at[idx_v.at[j]], rows_v.at[j], gsem).wait()

    def out_ref(blk):
        return out_hbm.at[pl.ds(wid * (_CPW // 128) + blk * _KR, _KR)]

    hash_block(0, idxs[0])
    fire(idxs[0], rows[0])
    for blk in range(1, _NB):
        b = blk & 1
        hash_block(blk, idxs[b])
        drain(idxs[1 - b], rows[1 - b])
        pltpu.async_copy(rows[1 - b], out_ref(blk - 1), osem)
        if blk >= 2:
            pltpu.make_async_copy(rows[b], out_ref(blk - 2), osem).wait()
        fire(idxs[b], rows[b])
    last = (_NB - 1) & 1
    drain(idxs[last], rows[last])
    pltpu.make_async_copy(rows[last], out_ref(_NB - 2), osem).wait()
    pltpu.sync_copy(rows[last], out_ref(_NB - 1))


def kernel(input_tensor, data, a, b):
    data_ext = jnp.concatenate([data, jnp.broadcast_to(data[-1], (_CHUNK,))])
    w = _build_w_sc(data_ext)
    ab = jnp.concatenate([a, b])
    out = _robe_sc(input_tensor, w, ab)
    return out.reshape(_BATCH, _FEAT, _NHASH * _CHUNK)


# speedup vs baseline: 5.5696x; 1.1165x over previous
import dataclasses
import functools

import jax
import jax.numpy as jnp
from jax import lax
from jax.experimental import pallas as pl
from jax.experimental.pallas import tpu as pltpu
from jax.experimental.pallas import tpu_sc as plsc

_SIZE = 1048576
_P = 2147483647
_CHUNK = 8
_NHASH = 8
_BATCH = 4096
_FEAT = 26
_M = _BATCH * _FEAT
_NCHUNKS = _M * _NHASH
_NW = 32
_CPW = _NCHUNKS // _NW
_RPW = _BATCH // _NW
_NB = 4
_RPB = _RPW // _NB
_K = _CPW // _NB
_KR = _K // 128
_GPR = _FEAT * _NHASH // 16
_WPW = _SIZE * _CHUNK // _NW
_NSB = 16
_SB = _WPW // _NSB
_SBR = _SB // _CHUNK

_CP = pltpu.CompilerParams()
if "needs_layout_passes" in pltpu.CompilerParams.__dataclass_fields__:
    _CP = dataclasses.replace(_CP, needs_layout_passes=False)
if "use_tc_tiling_on_sc" in pltpu.CompilerParams.__dataclass_fields__:
    _CP = dataclasses.replace(_CP, use_tc_tiling_on_sc=False)

_MESH = plsc.VectorSubcoreMesh(core_axis_name="c", subcore_axis_name="s")


@functools.partial(
    pl.kernel,
    out_type=jax.ShapeDtypeStruct((_SIZE, _CHUNK), jnp.float32),
    mesh=_MESH,
    scratch_types=[
        pltpu.VMEM((_SB + 8,), jnp.float32),
        pltpu.VMEM((_SB + 8,), jnp.float32),
        pltpu.VMEM((_SBR, _CHUNK), jnp.float32),
        pltpu.VMEM((_SBR, _CHUNK), jnp.float32),
        pltpu.SemaphoreType.DMA,
        pltpu.SemaphoreType.DMA,
    ],
    compiler_params=_CP,
)
def _build_w_sc(dext_hbm, w_hbm, buf_a, buf_b, sh_a, sh_b, isem, osem):
    wid = lax.axis_index("s") * 2 + lax.axis_index("c")
    s = wid >> 2
    q = wid & 3
    src0 = q * _WPW
    row0 = wid * (_WPW // _CHUNK)
    lane = lax.iota(jnp.int32, 16)
    rowpat = lane >> 3
    colpat = lane & 7
    bufs = (buf_a, buf_b)
    shs = (sh_a, sh_b)

    def src_ref(sb):
        return dext_hbm.at[pl.ds(src0 + sb * _SB, _SB + 8)]

    def dst_ref(sb):
        return w_hbm.at[pl.ds(row0 + sb * _SBR, _SBR)]

    pltpu.async_copy(src_ref(0), bufs[0], isem)
    for sb in range(_NSB):
        b = sb & 1
        pltpu.make_async_copy(src_ref(sb), bufs[b], isem).wait()
        if sb + 1 < _NSB:
            pltpu.async_copy(src_ref(sb + 1), bufs[1 - b], isem)
        if sb >= 2:
            pltpu.make_async_copy(shs[b], dst_ref(sb - 2), osem).wait()
        buf_v, sh_v = bufs[b], shs[b]

        @pl.loop(0, _SB // 64)
        def _shift(i):
            base = i * 64 + s
            r4 = 8 * i + rowpat
            for u in range(4):
                v = plsc.load_gather(buf_v, [(base + u * 16) + lane])
                plsc.store_scatter(sh_v, [r4 + 2 * u, colpat], v)

        pltpu.async_copy(sh_v, dst_ref(sb), osem)
    pltpu.make_async_copy(shs[0], dst_ref(_NSB - 2), osem).wait()
    pltpu.make_async_copy(shs[1], dst_ref(_NSB - 1), osem).wait()


@functools.partial(
    pl.kernel,
    out_type=jax.ShapeDtypeStruct((_NCHUNKS // 128, 128, _CHUNK), jnp.float32),
    mesh=_MESH,
    scratch_types=[
        pltpu.VMEM((_RPW, _FEAT), jnp.int32),
        pltpu.VMEM((16,), jnp.int32),
        pltpu.VMEM((_KR, 128), jnp.int32),
        pltpu.VMEM((_KR, 128), jnp.int32),
        pltpu.VMEM((_KR, 128, _CHUNK), jnp.float32),
        pltpu.VMEM((_KR, 128, _CHUNK), jnp.float32),
        pltpu.SemaphoreType.DMA,
        pltpu.SemaphoreType.DMA,
    ],
    compiler_params=_CP,
)
def _robe_sc(x_hbm, w_hbm, ab_hbm, out_hbm,
             x_v, ab_v, idx0, idx1, rows0, rows1, gsem, osem):
    wid = lax.axis_index("s") * 2 + lax.axis_index("c")
    pltpu.sync_copy(x_hbm.at[pl.ds(wid * _RPW, _RPW)], x_v)
    pltpu.sync_copy(ab_hbm, ab_v)
    lane = lax.iota(jnp.int32, 16)
    k8 = lane & 7
    av = plsc.load_gather(ab_v, [k8])
    bv = plsc.load_gather(ab_v, [k8 + 8])
    idxs = (idx0, idx1)
    rows = (rows0, rows1)

    def hash_block(blk, idx_v):
        @pl.loop(0, _RPB)
        def _row(r2):
            r = blk * _RPB + r2

            @pl.loop(0, _GPR)
            def _grp(g):
                col = (g * 16 + lane) >> 3
                xv = plsc.load_gather(x_v, [jnp.full((16,), r, jnp.int32), col])
                v = xv * av + bv
                v = jnp.where(v < 0, v + _P, v)
                v = jnp.where(v < 0, v + _P, v)
                v = jnp.where(v >= _P, v - _P, v)
                h = v & (_SIZE - 1)
                grp = r2 * _GPR + g
                idx_v[grp >> 3, pl.ds((grp & 7) * 16, 16)] = (
                    ((h & 7) << 17) + (h >> 3)
                )

    def fire(idx_v, rows_v):
        @pl.loop(0, _KR)
        def _f(j):
            pltpu.async_copy(w_hbm.at[idx_v.at[j]], rows_v.at[j], gsem)

    def drain(idx_v, rows_v):
        @pl.loop(0, _KR)
        def _d(j):
            pltpu.make_async_copy(w_hbm.at[idx_v.at[j]], rows_v.at[j], gsem).wait()

    def out_ref(blk):
        return out_hbm.at[pl.ds(wid * (_CPW // 128) + blk * _KR, _KR)]

    hash_block(0, idxs[0])
    fire(idxs[0], rows[0])
    for blk in range(1, _NB):
        b = blk & 1
        hash_block(blk, idxs[b])
        drain(idxs[1 - b], rows[1 - b])
        pltpu.async_copy(rows[1 - b], out_ref(blk - 1), osem)
        if blk >= 2:
            pltpu.make_async_copy(rows[b], out_ref(blk - 2), osem).wait()
        fire(idxs[b], rows[b])
    last = (_NB - 1) & 1
    drain(idxs[last], rows[last])
    pltpu.make_async_copy(rows[last], out_ref(_NB - 2), osem).wait()
    pltpu.sync_copy(rows[last], out_ref(_NB - 1))


def kernel(input_tensor, data, a, b):
    data_ext = jnp.concatenate([data, jnp.broadcast_to(data[-1], (_CHUNK,))])
    w = _build_w_sc(data_ext)
    ab = jnp.concatenate([a, b])
    out = _robe_sc(input_tensor, w, ab)
    return out.reshape(_BATCH, _FEAT, _NHASH * _CHUNK)
